# Initial kernel scaffold; baseline (speedup 1.0000x reference)
import jax
import jax.numpy as jnp
from jax.experimental import pallas as pl


def kernel(x_addr, x_tx, params, edge_index_input, edge_index_output, edge_index_spent):
    raise NotImplementedError("write your pallas kernel here")


# speedup vs baseline: 7.4936x; 7.4936x over previous
import functools
import math

import jax
import jax.numpy as jnp
from jax import lax
from jax.experimental import pallas as pl
from jax.experimental.pallas import tpu as pltpu
from jax.experimental.pallas import tpu_sc as plsc

H = 4
HID = 32
D = 8
NC = 2
NS = 16
CHUNK = 128
ROWB = 2000

_f32 = jnp.float32


def _ln_z(x):
    mu = jnp.mean(x, axis=-1, keepdims=True)
    xm = x - mu
    var = jnp.mean(xm * xm, axis=-1, keepdims=True)
    return xm * lax.rsqrt(var + 1e-5)


def _enc_body(xa_ref, wa_ref, ba_ref, xt_ref, wt_ref, bt_ref, oa_ref, ot_ref):
    oa_ref[...] = jnp.maximum(
        jnp.dot(xa_ref[...], wa_ref[...], preferred_element_type=_f32) + ba_ref[...], 0.0)
    ot_ref[...] = jnp.maximum(
        jnp.dot(xt_ref[...], wt_ref[...], preferred_element_type=_f32) + bt_ref[...], 0.0)


def _encode(x_addr, x_tx, wa, ba, wt, bt):
    n = x_addr.shape[0]
    fa, ft = x_addr.shape[1], x_tx.shape[1]
    full = lambda shp: pl.BlockSpec(shp, lambda i: (0, 0))
    return pl.pallas_call(
        _enc_body,
        grid=(n // ROWB,),
        in_specs=[pl.BlockSpec((ROWB, fa), lambda i: (i, 0)), full((fa, HID)), full((1, HID)),
                  pl.BlockSpec((ROWB, ft), lambda i: (i, 0)), full((ft, HID)), full((1, HID))],
        out_specs=[pl.BlockSpec((ROWB, HID), lambda i: (i, 0))] * 2,
        out_shape=[jax.ShapeDtypeStruct((n, HID), _f32)] * 2,
    )(x_addr, wa, ba, x_tx, wt, bt)


def _tables_body(xa_ref, xt_ref, ga, ba, gt, bt, wqa, bqa, wqt, bqt,
                 win, bin_, wout, bout, wsp, bsp,
                 xna, xnt, qa, qt, sin, sout, ssp):
    za = _ln_z(xa_ref[...])
    zt = _ln_z(xt_ref[...])
    xna[...] = za * ga[...] + ba[...]
    xnt[...] = zt * gt[...] + bt[...]
    qa[...] = jnp.dot(za, wqa[...], preferred_element_type=_f32) + bqa[...]
    qt[...] = jnp.dot(zt, wqt[...], preferred_element_type=_f32) + bqt[...]
    sin[...] = jnp.dot(za, win[...], preferred_element_type=_f32) + bin_[...]
    sout[...] = jnp.dot(zt, wout[...], preferred_element_type=_f32) + bout[...]
    ssp[...] = jnp.dot(zt, wsp[...], preferred_element_type=_f32) + bsp[...]


def _tables(xa, xt, fw):
    n = xa.shape[0]
    full = lambda shp: pl.BlockSpec(shp, lambda i: (0, 0))
    row = lambda w: pl.BlockSpec((ROWB, w), lambda i: (i, 0))
    return pl.pallas_call(
        _tables_body,
        grid=(n // ROWB,),
        in_specs=[row(HID), row(HID),
                  full((1, HID)), full((1, HID)), full((1, HID)), full((1, HID)),
                  full((HID, HID)), full((1, HID)), full((HID, HID)), full((1, HID)),
                  full((HID, 2 * HID)), full((1, 2 * HID)),
                  full((HID, 2 * HID)), full((1, 2 * HID)),
                  full((HID, 2 * HID)), full((1, 2 * HID))],
        out_specs=[row(HID), row(HID), row(HID), row(HID),
                   row(2 * HID), row(2 * HID), row(2 * HID)],
        out_shape=[jax.ShapeDtypeStruct((n, HID), _f32)] * 4
        + [jax.ShapeDtypeStruct((n, 2 * HID), _f32)] * 3,
    )(xa, xt, fw['g_addr'], fw['b_addr'], fw['g_tx'], fw['b_tx'],
      fw['wq_addr'], fw['bq_addr'], fw['wq_tx'], fw['bq_tx'],
      fw['wsrc_in'], fw['bsrc_in'], fw['wsrc_out'], fw['bsrc_out'],
      fw['wsrc_sp'], fw['bsrc_sp'])


def _rep8(x):
    return jnp.concatenate(
        [jnp.broadcast_to(x[:, h:h + 1], (x.shape[0], D)) for h in range(H)], axis=1)


def _gelu(x):
    return 0.5 * x * (1.0 + lax.erf(x * (1.0 / math.sqrt(2.0))))


def _combine_body(xa_ref, xt_ref, xna, xnt, pnin, pdin, pnout, pdout, pnsp, pdsp,
                  waa, baa, wat, bat, oa, ot):
    n_in = pnin[0] + pnin[1]
    d_in = pdin[0] + pdin[1]
    n_out = pnout[0] + pnout[1]
    d_out = pdout[0] + pdout[1]
    n_sp = pnsp[0] + pnsp[1]
    d_sp = pdsp[0] + pdsp[1]
    agg_t = (n_in / (_rep8(d_in) + 1e-16) + n_sp / (_rep8(d_sp) + 1e-16))
    agg_a = n_out / (_rep8(d_out) + 1e-16)
    o_a = jnp.dot(_gelu(agg_a), waa[...],
                  preferred_element_type=_f32) + baa[...]
    o_t = jnp.dot(_gelu(agg_t), wat[...],
                  preferred_element_type=_f32) + bat[...]
    oa[...] = xa_ref[...] + jnp.maximum(o_a + xna[...], 0.0)
    ot[...] = xt_ref[...] + jnp.maximum(o_t + xnt[...], 0.0)


def _combine(xa, xt, xna, xnt, p_in, p_out, p_sp, fw):
    n = xa.shape[0]
    full = lambda shp: pl.BlockSpec(shp, lambda i: (0, 0))
    row = lambda w: pl.BlockSpec((ROWB, w), lambda i: (i, 0))
    nspec = pl.BlockSpec((NC, ROWB, HID), lambda i: (0, i, 0))
    dspec = pl.BlockSpec((NC, ROWB, H), lambda i: (0, i, 0))
    return pl.pallas_call(
        _combine_body,
        grid=(n // ROWB,),
        in_specs=[row(HID), row(HID), row(HID), row(HID),
                  nspec, dspec, nspec, dspec, nspec, dspec,
                  full((HID, HID)), full((1, HID)), full((HID, HID)), full((1, HID))],
        out_specs=[row(HID), row(HID)],
        out_shape=[jax.ShapeDtypeStruct((n, HID), _f32)] * 2,
    )(xa, xt, xna, xnt, p_in[0], p_in[1], p_out[0], p_out[1], p_sp[0], p_sp[1],
      fw['wa_addr'], fw['ba_addr'], fw['wa_tx'], fw['ba_tx'])


def _final_body(x_ref, w_ref, b_ref, o_ref):
    o_ref[...] = jnp.dot(x_ref[...], w_ref[...], preferred_element_type=_f32) + b_ref[...]


def _final(x, w, b):
    n, width = x.shape[0], w.shape[1]
    full = lambda shp: pl.BlockSpec(shp, lambda i: (0, 0))
    return pl.pallas_call(
        _final_body,
        grid=(n // ROWB,),
        in_specs=[pl.BlockSpec((ROWB, HID), lambda i: (i, 0)),
                  full((HID, width)), full((1, width))],
        out_specs=pl.BlockSpec((ROWB, width), lambda i: (i, 0)),
        out_shape=jax.ShapeDtypeStruct((n, width), _f32),
    )(x, w, b)


def _round_up(x, m):
    return ((x + m - 1) // m) * m


@functools.cache
def _edge_pass(n_dst, e_pad, n_edges):
    rows_per = _round_up(-(-n_dst // NS), 8)
    n_pad = rows_per * NS
    per_w = e_pad // (NC * NS)
    n_chunks = per_w // CHUNK
    mesh = plsc.VectorSubcoreMesh(core_axis_name="c", subcore_axis_name="s",
                                  num_cores=NC, num_subcores=NS)

    def body(q_hbm, s_hbm, si_hbm, di_hbm, z2_hbm, z1_hbm, num_hbm, den_hbm,
             si_v, di_v, q_v, s_v, o_v, od0, od1, od2, od3,
             acc_num, ad0, ad1, ad2, ad3, sem1, sem2):
        cid = lax.axis_index("c")
        sid = lax.axis_index("s")
        wid = cid * NS + sid
        start = sid * rows_per
        ods = [od0, od1, od2, od3]
        ads = [ad0, ad1, ad2, ad3]
        pltpu.sync_copy(z2_hbm.at[pl.ds(start, rows_per)],
                        acc_num.at[pl.ds(start, rows_per)])
        for h in range(H):
            pltpu.sync_copy(z1_hbm.at[pl.ds(start, rows_per)],
                            ads[h].at[pl.ds(start, rows_per)])
        plsc.subcore_barrier()

        def chunk_body(t, carry):
            base = wid * per_w + t * CHUNK
            pltpu.sync_copy(si_hbm.at[pl.ds(base, CHUNK)], si_v)
            pltpu.sync_copy(di_hbm.at[pl.ds(base, CHUNK)], di_v)
            cp1 = pltpu.async_copy(q_hbm.at[di_v], q_v, sem1)
            cp2 = pltpu.async_copy(s_hbm.at[si_v], s_v, sem2)
            cp1.wait()
            cp2.wait()

            def grp(j, c2):
                rows = j * 16 + lax.iota(jnp.int32, 16)
                accs = [jnp.zeros((16,), _f32) for _ in range(H)]
                for f in range(HID):
                    col = jnp.full((16,), f, jnp.int32)
                    qv = plsc.load_gather(q_v, [rows, col])
                    kv = plsc.load_gather(s_v, [rows, col])
                    accs[f // D] = accs[f // D] + qv * kv
                valid = (base + rows) < n_edges
                ws = [jnp.where(valid, jnp.exp(a), 0.0) for a in accs]
                for h in range(H):
                    ods[h][pl.ds(j * 16, 16)] = ws[h]
                for f in range(HID):
                    vvv = plsc.load_gather(s_v, [rows, jnp.full((16,), HID + f, jnp.int32)])
                    plsc.store_scatter(o_v, [rows, jnp.full((16,), f, jnp.int32)],
                                       vvv * ws[f // D])
                return c2

            lax.fori_loop(0, CHUNK // 16, grp, 0)
            pltpu.sync_copy(o_v, acc_num.at[di_v], add=True)
            for h in range(H):
                pltpu.sync_copy(ods[h], ads[h].at[di_v], add=True)
            return carry

        lax.fori_loop(0, n_chunks, chunk_body, 0)
        plsc.subcore_barrier()
        pltpu.sync_copy(acc_num.at[pl.ds(start, rows_per)],
                        num_hbm.at[cid, pl.ds(start, rows_per)])
        for h in range(H):
            pltpu.sync_copy(ads[h].at[pl.ds(start, rows_per)],
                            den_hbm.at[cid, h, pl.ds(start, rows_per)])

    return pl.kernel(
        body,
        out_type=(jax.ShapeDtypeStruct((NC, n_pad, HID), _f32),
                  jax.ShapeDtypeStruct((NC, H, n_pad), _f32)),
        mesh=mesh,
        compiler_params=pltpu.CompilerParams(needs_layout_passes=False,
                                            use_tc_tiling_on_sc=False),
        scratch_types=[
            pltpu.VMEM((CHUNK,), jnp.int32),
            pltpu.VMEM((CHUNK,), jnp.int32),
            pltpu.VMEM((CHUNK, HID), _f32),
            pltpu.VMEM((CHUNK, 2 * HID), _f32),
            pltpu.VMEM((CHUNK, HID), _f32),
            pltpu.VMEM((CHUNK,), _f32),
            pltpu.VMEM((CHUNK,), _f32),
            pltpu.VMEM((CHUNK,), _f32),
            pltpu.VMEM((CHUNK,), _f32),
            pltpu.VMEM_SHARED((n_pad, HID), _f32),
            pltpu.VMEM_SHARED((n_pad,), _f32),
            pltpu.VMEM_SHARED((n_pad,), _f32),
            pltpu.VMEM_SHARED((n_pad,), _f32),
            pltpu.VMEM_SHARED((n_pad,), _f32),
            pltpu.SemaphoreType.DMA,
            pltpu.SemaphoreType.DMA,
        ],
    )


def _fold_conv(conv, norm):
    fw = {}
    for nt in ('addr', 'tx'):
        g = norm[nt]['g']
        b = norm[nt]['b']
        aa = jax.nn.sigmoid(conv['skip'][nt])
        fw[f'g_{nt}'] = ((1.0 - aa) * g).reshape(1, HID)
        fw[f'b_{nt}'] = ((1.0 - aa) * b).reshape(1, HID)
        wq = conv['q'][nt]['w']
        bq = conv['q'][nt]['b']
        fw[f'wq_{nt}'] = g[:, None] * wq
        fw[f'bq_{nt}'] = (b @ wq + bq).reshape(1, HID)
        fw[f'wa_{nt}'] = aa * conv['a'][nt]['w']
        fw[f'ba_{nt}'] = (aa * conv['a'][nt]['b']).reshape(1, HID)
    ets = ((('addr', 'input', 'tx'), 'in'),
           (('tx', 'output', 'addr'), 'out'),
           (('tx', 'spent_output', 'tx'), 'sp'))
    for et, tag in ets:
        src = et[0]
        es = '__'.join(et)
        g = norm[src]['g']
        b = norm[src]['b']
        wk = conv['k'][src]['w']
        bk = conv['k'][src]['b']
        wv = conv['v'][src]['w']
        bv = conv['v'][src]['b']
        a_rel = conv['a_rel'][es]
        m_rel = conv['m_rel'][es]
        scale = jnp.repeat(conv['p_rel'][es] / math.sqrt(D), D)
        wkk = jnp.einsum('ihd,hde->ihe', (g[:, None] * wk).reshape(HID, H, D),
                         a_rel).reshape(HID, HID) * scale
        bkk = jnp.einsum('hd,hde->he', (b @ wk + bk).reshape(H, D),
                         a_rel).reshape(HID) * scale
        wvv = jnp.einsum('ihd,hde->ihe', (g[:, None] * wv).reshape(HID, H, D),
                         m_rel).reshape(HID, HID)
        bvv = jnp.einsum('hd,hde->he', (b @ wv + bv).reshape(H, D),
                         m_rel).reshape(HID)
        fw[f'wsrc_{tag}'] = jnp.concatenate([wkk, wvv], axis=1)
        fw[f'bsrc_{tag}'] = jnp.concatenate([bkk, bvv]).reshape(1, 2 * HID)
    return fw


def _pad_edges(ei):
    e = ei.shape[1]
    cap = NC * NS * CHUNK
    e_pad = ((e + cap - 1) // cap) * cap
    si = jnp.concatenate([ei[0].astype(jnp.int32), jnp.zeros((e_pad - e,), jnp.int32)])
    di = jnp.concatenate([ei[1].astype(jnp.int32), jnp.zeros((e_pad - e,), jnp.int32)])
    return si, di, e_pad, e


def kernel(x_addr, x_tx, params, edge_index_input, edge_index_output,
           edge_index_spent):
    n = x_addr.shape[0]
    edges = {tag: _pad_edges(ei) for tag, ei in
             (('in', edge_index_input), ('out', edge_index_output),
              ('sp', edge_index_spent))}
    n_pad = _round_up(-(-n // NS), 8) * NS
    zeros2 = jnp.zeros((n_pad, HID), _f32)
    zeros1 = jnp.zeros((n_pad,), _f32)
    enc = params['enc']
    xa, xt = _encode(x_addr, x_tx,
                     enc['addr']['w'], enc['addr']['b'].reshape(1, HID),
                     enc['tx']['w'], enc['tx']['b'].reshape(1, HID))
    for conv in params['convs']:
        fw = _fold_conv(conv, params['norm'])
        xna, xnt, qa, qt, s_in, s_out, s_sp = _tables(xa, xt, fw)
        tabs = {'in': (qt, s_in), 'out': (qa, s_out), 'sp': (qt, s_sp)}
        parts = {}
        for tag in ('in', 'out', 'sp'):
            q_tab, s_tab = tabs[tag]
            si, di, e_pad, e = edges[tag]
            num, den = _edge_pass(n, e_pad, e)(q_tab, s_tab, si, di,
                                               zeros2, zeros1)
            parts[tag] = (num, jnp.swapaxes(den, 1, 2))
        xa, xt = _combine(xa, xt, xna, xnt, parts['in'], parts['out'],
                          parts['sp'], fw)
    return _final(xa, params['lin']['w'],
                  params['lin']['b'].reshape(1, params['lin']['w'].shape[1]))
